# baseline (speedup 1.0000x reference)
import jax
import jax.numpy as jnp
from jax import lax
from jax.experimental import pallas as pl
from jax.experimental.pallas import tpu as pltpu
from jax.experimental.pallas import tpu_sc as plsc

B, P, F = 16384, 676, 26
N = B * P
NC, NS, L = 2, 16, 16
NW = NC * NS
PER_W = N // NW
CH = 6656
NCH = PER_W // CH
TAB = 688


def _sc_body(v_hbm, lg_hbm, out_hbm, tab, vb0, ob0, vb1, ob1,
             si0, si1, so0, so1):
    pltpu.sync_copy(lg_hbm, tab)

    @plsc.parallel_loop(0, TAB, step=L)
    def _(t):
        sl = pl.ds(t, L)
        tab[sl] = 1.0 / (1.0 + jnp.exp(-tab[sl]))

    wid = lax.axis_index("s") * NC + lax.axis_index("c")
    base = wid * PER_W

    ins = ((vb0, si0), (vb1, si1))
    outs = ((ob0, so0), (ob1, so1))

    def start_in(b, c):
        vb, sem = ins[b]
        pltpu.async_copy(v_hbm.at[pl.ds(base + c * CH, CH)], vb, sem)

    def wait_in(b):
        vb, sem = ins[b]
        pltpu.make_async_copy(v_hbm.at[pl.ds(0, CH)], vb, sem).wait()

    def start_out(b, c):
        ob, sem = outs[b]
        pltpu.async_copy(ob, out_hbm.at[pl.ds(base + c * CH, CH)], sem)

    def wait_out(b):
        ob, sem = outs[b]
        pltpu.make_async_copy(ob, out_hbm.at[pl.ds(0, CH)], sem).wait()

    def compute(b):
        vb, _ = ins[b]
        ob, _ = outs[b]

        @plsc.parallel_loop(0, CH, step=L, unroll=8)
        def _(i):
            sl = pl.ds(i, L)
            v = vb[sl]
            idx = lax.shift_right_logical(v, 22)
            d = plsc.bitcast(lax.shift_left(v, 10), jnp.float32)
            g = plsc.load_gather(tab, [idx])
            ob[sl] = g * jnp.exp(-d)

    start_in(0, 0)

    def group(gg, carry):
        for b in range(2):
            c = gg * 2 + b

            @pl.when(c + 1 < NCH)
            def _():
                start_in(1 - b, c + 1)

            wait_in(b)

            @pl.when(gg > 0)
            def _():
                wait_out(b)

            compute(b)
            start_out(b, c)
        return carry

    lax.fori_loop(0, NCH // 2, group, 0)
    wait_out(0)
    wait_out(1)


@jax.jit
def kernel(pos_i, pos_j, distance, logits):
    pi = pos_i.astype(jnp.int32)
    pj = pos_j.astype(jnp.int32)
    dbits = lax.bitcast_convert_type(distance, jnp.int32)
    v = ((pi * F + pj) << 22) | lax.shift_right_logical(dbits, 10)
    v = v.reshape(N)
    lg = jnp.pad(logits.reshape(F * F), (0, TAB - F * F))

    mesh = plsc.VectorSubcoreMesh(
        core_axis_name="c", subcore_axis_name="s", num_cores=NC, num_subcores=NS
    )
    out = pl.kernel(
        _sc_body,
        out_type=jax.ShapeDtypeStruct((N,), jnp.float32),
        mesh=mesh,
        scratch_types=[
            pltpu.VMEM((TAB,), jnp.float32),
            pltpu.VMEM((CH,), jnp.int32),
            pltpu.VMEM((CH,), jnp.float32),
            pltpu.VMEM((CH,), jnp.int32),
            pltpu.VMEM((CH,), jnp.float32),
            pltpu.SemaphoreType.DMA,
            pltpu.SemaphoreType.DMA,
            pltpu.SemaphoreType.DMA,
            pltpu.SemaphoreType.DMA,
        ],
        compiler_params=pltpu.CompilerParams(needs_layout_passes=False),
    )(v, lg)
    return out.reshape(B, P)


# speedup vs baseline: 1.0568x; 1.0568x over previous
import jax
import jax.numpy as jnp
from jax import lax
from jax.experimental import pallas as pl
from jax.experimental.pallas import tpu as pltpu
from jax.experimental.pallas import tpu_sc as plsc

B, P, F = 16384, 676, 26
NC, NS, L = 2, 16, 16
NW = NC * NS
ROWS_W = B // NW
RWS = 16
NCH = ROWS_W // RWS
NFULL = (P // L) * L
TAIL = P - L
TAB = 688


def _sc_body(v_hbm, lg_hbm, out_hbm, tab, vb0, ob0, vb1, ob1,
             si0, si1, so0, so1):
    pltpu.sync_copy(lg_hbm, tab)

    @plsc.parallel_loop(0, TAB, step=L)
    def _(t):
        sl = pl.ds(t, L)
        tab[sl] = 1.0 / (1.0 + jnp.exp(-tab[sl]))

    wid = lax.axis_index("s") * NC + lax.axis_index("c")
    base = wid * ROWS_W

    ins = ((vb0, si0), (vb1, si1))
    outs = ((ob0, so0), (ob1, so1))

    def start_in(b, c):
        vb, sem = ins[b]
        pltpu.async_copy(v_hbm.at[pl.ds(base + c * RWS, RWS)], vb, sem)

    def wait_in(b):
        vb, sem = ins[b]
        pltpu.make_async_copy(v_hbm.at[pl.ds(0, RWS)], vb, sem).wait()

    def start_out(b, c):
        ob, sem = outs[b]
        pltpu.async_copy(ob, out_hbm.at[pl.ds(base + c * RWS, RWS)], sem)

    def wait_out(b):
        ob, sem = outs[b]
        pltpu.make_async_copy(ob, out_hbm.at[pl.ds(0, RWS)], sem).wait()

    def compute(b):
        vb, _ = ins[b]
        ob, _ = outs[b]

        def row(r, carry):
            def cell(i):
                sl = pl.ds(i, L)
                v = vb[r, sl]
                idx = lax.shift_right_logical(v, 22)
                d = plsc.bitcast(lax.shift_left(v, 10), jnp.float32)
                g = plsc.load_gather(tab, [idx])
                ob[r, sl] = g * jnp.exp(-d)

            loop = plsc.parallel_loop(0, NFULL, step=L, unroll=7)
            loop(cell)
            cell(TAIL)
            return carry

        lax.fori_loop(0, RWS, row, 0)

    start_in(0, 0)

    def group(gg, carry):
        for b in range(2):
            c = gg * 2 + b

            @pl.when(c + 1 < NCH)
            def _():
                start_in(1 - b, c + 1)

            wait_in(b)

            @pl.when(gg > 0)
            def _():
                wait_out(b)

            compute(b)
            start_out(b, c)
        return carry

    lax.fori_loop(0, NCH // 2, group, 0)
    wait_out(0)
    wait_out(1)


@jax.jit
def kernel(pos_i, pos_j, distance, logits):
    pi = pos_i.astype(jnp.int32)
    pj = pos_j.astype(jnp.int32)
    dbits = lax.bitcast_convert_type(distance, jnp.int32)
    v = ((pi * F + pj) << 22) | lax.shift_right_logical(dbits, 10)
    lg = jnp.pad(logits.reshape(F * F), (0, TAB - F * F))

    mesh = plsc.VectorSubcoreMesh(
        core_axis_name="c", subcore_axis_name="s", num_cores=NC, num_subcores=NS
    )
    return pl.kernel(
        _sc_body,
        out_type=jax.ShapeDtypeStruct((B, P), jnp.float32),
        mesh=mesh,
        scratch_types=[
            pltpu.VMEM((TAB,), jnp.float32),
            pltpu.VMEM((RWS, P), jnp.int32),
            pltpu.VMEM((RWS, P), jnp.float32),
            pltpu.VMEM((RWS, P), jnp.int32),
            pltpu.VMEM((RWS, P), jnp.float32),
            pltpu.SemaphoreType.DMA,
            pltpu.SemaphoreType.DMA,
            pltpu.SemaphoreType.DMA,
            pltpu.SemaphoreType.DMA,
        ],
        compiler_params=pltpu.CompilerParams(needs_layout_passes=False),
    )(v, lg)
